# Initial kernel scaffold; baseline (speedup 1.0000x reference)
import jax
import jax.numpy as jnp
from jax.experimental import pallas as pl


def kernel(x, edge_index, x0, bridge_index, W1, b1, g1, be1, W2, b2, g2, be2, eps):
    raise NotImplementedError("write your pallas kernel here")


# speedup vs baseline: 4.4193x; 4.4193x over previous
import functools

import jax
import jax.numpy as jnp
from jax import lax
from jax.experimental import pallas as pl
from jax.experimental.pallas import tpu as pltpu
from jax.experimental.pallas import tpu_sc as plsc

_NC = 2
_NS = 16
_C = 128
_LANES = 16


def _sc_edge_agg(x, x0, srcp, dstp, brp, n_pad, nch):
    n, d = x.shape
    nw = _NC * _NS
    kd = d // _LANES
    rpt = (n // _NS) // 8 * 8
    last = n - (_NS - 1) * rpt
    zch = n_pad // _NS // _C

    mesh = plsc.VectorSubcoreMesh(
        core_axis_name="c", subcore_axis_name="s",
        num_cores=_NC, num_subcores=_NS)

    @functools.partial(
        pl.kernel,
        out_type=jax.ShapeDtypeStruct((_NC, n, d), jnp.float32),
        mesh=mesh,
        scratch_types=[
            pltpu.VMEM_SHARED((n_pad, d), jnp.float32),
            pltpu.VMEM((1, _C), jnp.int32),
            pltpu.VMEM((1, _C), jnp.int32),
            pltpu.VMEM((1, _C), jnp.int32),
            pltpu.VMEM((_C, d), jnp.float32),
            pltpu.VMEM((_C, d), jnp.float32),
            pltpu.SemaphoreType.DMA,
            pltpu.SemaphoreType.DMA,
        ],
    )
    def body(x_hbm, x0_hbm, src_hbm, dst_hbm, br_hbm, out_hbm,
             acc_sh, src_v, dst_v, br_v, xbuf, ybuf, sem0, sem1):
        c = lax.axis_index("c")
        s = lax.axis_index("s")
        w = c * _NS + s

        def zrow(r, carry):
            for k in range(kd):
                xbuf[r, pl.ds(k * _LANES, _LANES)] = jnp.zeros(
                    (_LANES,), jnp.float32)
            return carry
        lax.fori_loop(0, _C, zrow, 0)
        base = s * (n_pad // _NS)
        for k in range(zch):
            pltpu.sync_copy(xbuf, acc_sh.at[pl.ds(base + k * _C, _C)])
        plsc.subcore_barrier()

        def chunk(j, carry):
            pltpu.sync_copy(src_hbm.at[w, pl.ds(j, 1)], src_v)
            pltpu.sync_copy(dst_hbm.at[w, pl.ds(j, 1)], dst_v)
            pltpu.sync_copy(br_hbm.at[w, pl.ds(j, 1)], br_v)
            cp0 = pltpu.async_copy(x_hbm.at[src_v.at[0]], xbuf, sem0)
            cp1 = pltpu.async_copy(x0_hbm.at[br_v.at[0]], ybuf, sem1)
            cp0.wait()
            cp1.wait()

            def row(r, rc):
                for k in range(kd):
                    sl = pl.ds(k * _LANES, _LANES)
                    xbuf[r, sl] = jnp.maximum(xbuf[r, sl] + ybuf[r, sl], 0.0)
                return rc
            lax.fori_loop(0, _C, row, 0)
            pltpu.sync_copy(xbuf, acc_sh.at[dst_v.at[0]], add=True)
            return carry
        lax.fori_loop(0, nch, chunk, 0)

        plsc.subcore_barrier()

        @pl.when(s < _NS - 1)
        def _():
            pltpu.sync_copy(acc_sh.at[pl.ds(s * rpt, rpt)],
                            out_hbm.at[c, pl.ds(s * rpt, rpt)])

        @pl.when(s == _NS - 1)
        def _():
            pltpu.sync_copy(acc_sh.at[pl.ds((_NS - 1) * rpt, last)],
                            out_hbm.at[c, pl.ds((_NS - 1) * rpt, last)])

    return body(x, x0, srcp, dstp, brp)


def _mlp_stage1(x, aggs, w1, b1, eps):
    n, d = x.shape
    h = w1.shape[1]
    blk = 2000
    nblk = n // blk

    def body(x_ref, agg_ref, w1_ref, b1_ref, eps_ref, y_ref):
        hblk = ((1.0 + eps_ref[0, 0]) * x_ref[...]
                + agg_ref[0] + agg_ref[1])
        y_ref[...] = jnp.dot(hblk, w1_ref[...],
                             preferred_element_type=jnp.float32) + b1_ref[...]

    return pl.pallas_call(
        body,
        grid=(nblk,),
        in_specs=[
            pl.BlockSpec((blk, d), lambda i: (i, 0)),
            pl.BlockSpec((_NC, blk, d), lambda i: (0, i, 0)),
            pl.BlockSpec((d, h), lambda i: (0, 0)),
            pl.BlockSpec((1, h), lambda i: (0, 0)),
            pl.BlockSpec(memory_space=pltpu.SMEM),
        ],
        out_specs=pl.BlockSpec((blk, h), lambda i: (i, 0)),
        out_shape=jax.ShapeDtypeStruct((n, h), jnp.float32),
    )(x, aggs, w1, b1, eps)


def _mlp_stage2(y, g1, be1, w2, b2, g2, be2):
    def body(y_ref, g1_ref, be1_ref, w2_ref, b2_ref, g2_ref, be2_ref, o_ref):
        y = y_ref[...]
        m1 = jnp.mean(y, axis=0, keepdims=True)
        v1 = jnp.mean((y - m1) ** 2, axis=0, keepdims=True)
        y = g1_ref[...] * (y - m1) / jnp.sqrt(v1 + 1e-5) + be1_ref[...]
        y = jnp.maximum(y, 0.0)
        z = jnp.dot(y, w2_ref[...],
                    preferred_element_type=jnp.float32) + b2_ref[...]
        m2 = jnp.mean(z, axis=0, keepdims=True)
        v2 = jnp.mean((z - m2) ** 2, axis=0, keepdims=True)
        z = g2_ref[...] * (z - m2) / jnp.sqrt(v2 + 1e-5) + be2_ref[...]
        o_ref[...] = jnp.maximum(z, 0.0)

    n, h = y.shape
    return pl.pallas_call(
        body,
        out_shape=jax.ShapeDtypeStruct((n, h), jnp.float32),
    )(y, g1, be1, w2, b2, g2, be2)


def kernel(x, edge_index, x0, bridge_index, W1, b1, g1, be1, W2, b2, g2, be2,
           eps):
    n, d = x.shape
    e = bridge_index.shape[0]
    h = W1.shape[1]
    nw = _NC * _NS

    epw = -(-e // (nw * _C)) * _C
    e_pad = epw * nw
    nch = epw // _C
    pad = e_pad - e
    src = edge_index[0]
    dst = edge_index[1]
    if pad:
        zpad = jnp.zeros((pad,), jnp.int32)
        src = jnp.concatenate([src, zpad])
        dst = jnp.concatenate([dst, jnp.full((pad,), n, jnp.int32)])
        bridge_index = jnp.concatenate([bridge_index, zpad])
    srcp = src.reshape(nw, nch, _C)
    dstp = dst.reshape(nw, nch, _C)
    brp = bridge_index.reshape(nw, nch, _C)

    n_pad = -(-(n + 1) // (_NS * _C)) * (_NS * _C)

    aggs = _sc_edge_agg(x, x0, srcp, dstp, brp, n_pad, nch)

    y = _mlp_stage1(x, aggs, W1, b1.reshape(1, h), eps.reshape(1, 1))
    return _mlp_stage2(y, g1.reshape(1, h), be1.reshape(1, h), W2,
                       b2.reshape(1, h), g2.reshape(1, h), be2.reshape(1, h))


# speedup vs baseline: 5.8149x; 1.3158x over previous
import functools

import jax
import jax.numpy as jnp
from jax import lax
from jax.experimental import pallas as pl
from jax.experimental.pallas import tpu as pltpu
from jax.experimental.pallas import tpu_sc as plsc

_NC = 2
_NS = 16
_C = 96
_NBUF = 3
_LANES = 16


def _sc_edge_agg(x, x0, idxp, n_pad, nch):
    n, d = x.shape
    kd = d // _LANES
    rpt = (n // _NS) // 8 * 8
    last = n - (_NS - 1) * rpt
    ng = nch // _NBUF

    mesh = plsc.VectorSubcoreMesh(
        core_axis_name="c", subcore_axis_name="s",
        num_cores=_NC, num_subcores=_NS)

    @functools.partial(
        pl.kernel,
        out_type=jax.ShapeDtypeStruct((_NC, n, d), jnp.float32),
        mesh=mesh,
        scratch_types=(
            [
                pltpu.VMEM_SHARED((n_pad, d), jnp.float32),
                pltpu.VMEM((_NBUF, _C, d), jnp.float32),
                pltpu.VMEM((2, _NBUF, 3, _C), jnp.int32),
            ]
            + [pltpu.SemaphoreType.DMA] * (3 * _NBUF + 2)
        ),
    )
    def body(x_hbm, x0_hbm, idx_hbm, out_hbm, acc_sh, xb, ib, *sems):
        sem_a = sems[0:_NBUF]
        sem_b = sems[_NBUF:2 * _NBUF]
        sem_c = sems[2 * _NBUF:3 * _NBUF]
        sem_i = sems[3 * _NBUF:]
        c = lax.axis_index("c")
        s = lax.axis_index("s")
        w = c * _NS + s

        def zrow(r, carry):
            for k in range(kd):
                xb[0, r, pl.ds(k * _LANES, _LANES)] = jnp.zeros(
                    (_LANES,), jnp.float32)
            return carry
        lax.fori_loop(0, _C, zrow, 0)
        zrows = n_pad // _NS
        base = s * zrows
        for k in range(zrows // _C):
            pltpu.sync_copy(xb.at[0], acc_sh.at[pl.ds(base + k * _C, _C)])
        zrem = zrows - (zrows // _C) * _C
        if zrem:
            pltpu.sync_copy(xb.at[0, pl.ds(0, zrem)],
                            acc_sh.at[pl.ds(base + zrows - zrem, zrem)])
        plsc.subcore_barrier()

        def relu_buf(b):
            def row(r, rc):
                for k in range(kd):
                    sl = pl.ds(k * _LANES, _LANES)
                    xb[b, r, sl] = jnp.maximum(xb[b, r, sl], 0.0)
                return rc
            lax.fori_loop(0, _C, row, 0)

        def load_idx_group(g, q, sem):
            return pltpu.async_copy(
                idx_hbm.at[w, pl.ds(g * _NBUF, _NBUF)], ib.at[q], sem)

        def issue_gather(g, q, b):
            return pltpu.async_copy(
                x_hbm.at[ib.at[q, b, 0]], xb.at[b], sem_a[b])

        def issue_gather_add(q, b):
            return pltpu.async_copy(
                x0_hbm.at[ib.at[q, b, 2]], xb.at[b], sem_b[b], add=True)

        def issue_scatter(q, b):
            return pltpu.async_copy(
                xb.at[b], acc_sh.at[ib.at[q, b, 1]], sem_c[b], add=True)

        idx_d = [None, None]
        sc_d = [None] * _NBUF
        load_idx_group(0, 0, sem_i[0]).wait()
        if ng > 1:
            idx_d[1] = load_idx_group(1, 1, sem_i[1])
        for g in range(ng):
            q = g % 2
            if g >= 1:
                idx_d[q].wait()
            if 1 <= g + 1 < ng:
                idx_d[1 - q] = load_idx_group(g + 1, 1 - q, sem_i[1 - q])
            gx = []
            for b in range(_NBUF):
                if sc_d[b] is not None:
                    sc_d[b].wait()
                gx.append(issue_gather(g, q, b))
            ga = []
            for b in range(_NBUF):
                gx[b].wait()
                ga.append(issue_gather_add(q, b))
            for b in range(_NBUF):
                ga[b].wait()
                relu_buf(b)
                sc_d[b] = issue_scatter(q, b)
        for b in range(_NBUF):
            sc_d[b].wait()

        plsc.subcore_barrier()

        @pl.when(s < _NS - 1)
        def _():
            pltpu.sync_copy(acc_sh.at[pl.ds(s * rpt, rpt)],
                            out_hbm.at[c, pl.ds(s * rpt, rpt)])

        @pl.when(s == _NS - 1)
        def _():
            pltpu.sync_copy(acc_sh.at[pl.ds((_NS - 1) * rpt, last)],
                            out_hbm.at[c, pl.ds((_NS - 1) * rpt, last)])

    return body(x, x0, idxp)


def _mlp_stage1(x, aggs, w1, b1, eps):
    n, d = x.shape
    h = w1.shape[1]
    blk = 2000
    nblk = n // blk

    def body(x_ref, agg_ref, w1_ref, b1_ref, eps_ref, y_ref):
        hblk = ((1.0 + eps_ref[0, 0]) * x_ref[...]
                + agg_ref[0] + agg_ref[1])
        y_ref[...] = jnp.dot(hblk, w1_ref[...],
                             preferred_element_type=jnp.float32) + b1_ref[...]

    return pl.pallas_call(
        body,
        grid=(nblk,),
        in_specs=[
            pl.BlockSpec((blk, d), lambda i: (i, 0)),
            pl.BlockSpec((_NC, blk, d), lambda i: (0, i, 0)),
            pl.BlockSpec((d, h), lambda i: (0, 0)),
            pl.BlockSpec((1, h), lambda i: (0, 0)),
            pl.BlockSpec(memory_space=pltpu.SMEM),
        ],
        out_specs=pl.BlockSpec((blk, h), lambda i: (i, 0)),
        out_shape=jax.ShapeDtypeStruct((n, h), jnp.float32),
    )(x, aggs, w1, b1, eps)


def _mlp_stage2(y, g1, be1, w2, b2, g2, be2):
    def body(y_ref, g1_ref, be1_ref, w2_ref, b2_ref, g2_ref, be2_ref, o_ref):
        y = y_ref[...]
        m1 = jnp.mean(y, axis=0, keepdims=True)
        v1 = jnp.mean((y - m1) ** 2, axis=0, keepdims=True)
        y = g1_ref[...] * (y - m1) / jnp.sqrt(v1 + 1e-5) + be1_ref[...]
        y = jnp.maximum(y, 0.0)
        z = jnp.dot(y, w2_ref[...],
                    preferred_element_type=jnp.float32) + b2_ref[...]
        m2 = jnp.mean(z, axis=0, keepdims=True)
        v2 = jnp.mean((z - m2) ** 2, axis=0, keepdims=True)
        z = g2_ref[...] * (z - m2) / jnp.sqrt(v2 + 1e-5) + be2_ref[...]
        o_ref[...] = jnp.maximum(z, 0.0)

    n, h = y.shape
    return pl.pallas_call(
        body,
        out_shape=jax.ShapeDtypeStruct((n, h), jnp.float32),
    )(y, g1, be1, w2, b2, g2, be2)


def kernel(x, edge_index, x0, bridge_index, W1, b1, g1, be1, W2, b2, g2, be2,
           eps):
    n, d = x.shape
    e = bridge_index.shape[0]
    h = W1.shape[1]
    nw = _NC * _NS

    epw = -(-e // (nw * _C * _NBUF)) * _C * _NBUF
    e_pad = epw * nw
    nch = epw // _C
    pad = e_pad - e
    src = edge_index[0]
    dst = edge_index[1]
    if pad:
        zpad = jnp.zeros((pad,), jnp.int32)
        src = jnp.concatenate([src, zpad])
        dst = jnp.concatenate([dst, jnp.full((pad,), n, jnp.int32)])
        bridge_index = jnp.concatenate([bridge_index, zpad])
    idxp = jnp.stack(
        [src.reshape(nw, nch, _C), dst.reshape(nw, nch, _C),
         bridge_index.reshape(nw, nch, _C)], axis=2)

    n_pad = -(-(n + 1) // (_NS * 8)) * (_NS * 8)

    aggs = _sc_edge_agg(x, x0, idxp, n_pad, nch)

    y = _mlp_stage1(x, aggs, W1, b1.reshape(1, h), eps.reshape(1, 1))
    return _mlp_stage2(y, g1.reshape(1, h), be1.reshape(1, h), W2,
                       b2.reshape(1, h), g2.reshape(1, h), be2.reshape(1, h))
